# Initial kernel scaffold; baseline (speedup 1.0000x reference)
import jax
import jax.numpy as jnp
from jax.experimental import pallas as pl


def kernel(inp, edge_index, weights, bias):
    raise NotImplementedError("write your pallas kernel here")


# speedup vs baseline: 5.3049x; 5.3049x over previous
import functools

import jax
import jax.numpy as jnp
from jax import lax
from jax.experimental import pallas as pl
from jax.experimental.pallas import tpu as pltpu
from jax.experimental.pallas import tpu_sc as plsc

N = 10000
E = 320000
R = 4
IN_SIZE = 128
OUT_SIZE = 128

NC = 2
NS = 16
NW = NC * NS

CH = 128
TOTAL_E = R * E
CHUNKS_PER_W = -(-TOTAL_E // (NW * CH))
PW = CHUNKS_PER_W * CH
TP = PW * NW
NPAD = 10112
DUMMY_DST = 10000
ZROWS = NPAD // NS


def _h_body(x_ref, w_ref, h_ref):
    h_ref[...] = jnp.dot(x_ref[...], w_ref[0], preferred_element_type=jnp.float32)


def _combine_body(p_ref, b_ref, o_ref):
    bias_sum = jnp.sum(b_ref[...], axis=0, keepdims=True)
    o_ref[...] = p_ref[0, :N] + p_ref[1, :N] + bias_sum


def _sc_agg(h, src, dst):
    mesh = plsc.VectorSubcoreMesh(core_axis_name="c", subcore_axis_name="s")

    @functools.partial(
        pl.kernel,
        mesh=mesh,
        out_type=jax.ShapeDtypeStruct((NC, NPAD, OUT_SIZE), jnp.float32),
        scratch_types=[
            pltpu.VMEM((CH,), jnp.int32),
            pltpu.VMEM((CH,), jnp.int32),
            pltpu.VMEM((CH, OUT_SIZE), jnp.float32),
            pltpu.VMEM_SHARED((NPAD, OUT_SIZE), jnp.float32),
            pltpu.SemaphoreType.DMA,
        ],
    )
    def sc_fn(h_hbm, src_hbm, dst_hbm, part_hbm, src_v, dst_v, rows_v, acc, sem):
        cid = lax.axis_index("c")
        sid = lax.axis_index("s")

        @pl.loop(0, CH)
        def _(i):
            @pl.loop(0, OUT_SIZE, step=16)
            def _(j):
                rows_v[i, pl.ds(j, 16)] = jnp.zeros((16,), jnp.float32)

        zbase = sid * ZROWS
        @pl.loop(0, ZROWS // CH)
        def _(k):
            pltpu.sync_copy(rows_v, acc.at[pl.ds(zbase + k * CH, CH)])
        pltpu.sync_copy(rows_v.at[pl.ds(0, ZROWS % CH)],
                        acc.at[pl.ds(zbase + (ZROWS // CH) * CH, ZROWS % CH)])
        plsc.subcore_barrier()

        ebase = (cid * NS + sid) * PW

        @pl.loop(0, CHUNKS_PER_W)
        def _(ch):
            off = ebase + ch * CH
            pltpu.sync_copy(src_hbm.at[pl.ds(off, CH)], src_v)
            pltpu.sync_copy(dst_hbm.at[pl.ds(off, CH)], dst_v)
            pltpu.async_copy(h_hbm.at[src_v], rows_v, sem).wait()
            pltpu.sync_copy(rows_v, acc.at[dst_v], add=True)

        plsc.subcore_barrier()
        pltpu.sync_copy(acc.at[pl.ds(sid * ZROWS, ZROWS)],
                        part_hbm.at[cid, pl.ds(sid * ZROWS, ZROWS)])

    return sc_fn(h, src, dst)


def kernel(inp, edge_index, weights, bias):
    h = pl.pallas_call(
        _h_body,
        grid=(R, N // 1000),
        in_specs=[
            pl.BlockSpec((1000, IN_SIZE), lambda r, i: (i, 0)),
            pl.BlockSpec((1, IN_SIZE, OUT_SIZE), lambda r, i: (r, 0, 0)),
        ],
        out_specs=pl.BlockSpec((1000, OUT_SIZE), lambda r, i: (r * (N // 1000) + i, 0)),
        out_shape=jax.ShapeDtypeStruct((R * N, OUT_SIZE), jnp.float32),
    )(inp, weights)

    rel_off = (jnp.arange(R, dtype=jnp.int32) * N)[:, None]
    src_flat = (edge_index[:, 1, :] + rel_off).reshape(-1)
    dst_flat = edge_index[:, 0, :].reshape(-1)
    pad = TP - TOTAL_E
    src_flat = jnp.concatenate([src_flat, jnp.zeros((pad,), jnp.int32)])
    dst_flat = jnp.concatenate([dst_flat, jnp.full((pad,), DUMMY_DST, jnp.int32)])

    part = _sc_agg(h, src_flat, dst_flat)

    out = pl.pallas_call(
        _combine_body,
        out_shape=jax.ShapeDtypeStruct((N, OUT_SIZE), jnp.float32),
    )(part, bias)
    return out
